# baseline (speedup 1.0000x reference)
import jax
import jax.numpy as jnp
from jax.experimental import pallas as pl
from jax.experimental.pallas import tpu as pltpu

_EPS = 1e-12


def _make_kernel(B, N, M, D, TN):
    NI = N // TN

    LT = M // 128

    def _chamfer_kernel(x_ref, t_ref, out_ref, colacc_ref, t2s_ref, rowbuf_ref):
        b = pl.program_id(0)
        i = pl.program_id(1)
        first = jnp.logical_and(b == 0, i == 0)
        last = jnp.logical_and(b == B - 1, i == NI - 1)
        xb = x_ref[0]

        @pl.when(first)
        def _():
            out_ref[...] = jnp.zeros_like(out_ref)

        @pl.when(jnp.logical_not(first))
        def _():
            rmin = jnp.min(rowbuf_ref[...], axis=1, keepdims=True)
            out_ref[...] += jnp.sum(jnp.sqrt(jnp.maximum(rmin, _EPS))) / (N * B)

        @pl.when(i == 0)
        def _():
            tb0 = t_ref[0]
            t2s_ref[...] = jnp.sum(tb0 * tb0, axis=0, keepdims=True)

        tb = t_ref[0]
        xb2 = xb + xb
        x2s = jnp.sum(xb * xb, axis=1, keepdims=True)

        cross2 = jax.lax.dot_general(
            xb2, tb, (((1,), (0,)), ((), ())),
            precision=jax.lax.Precision.DEFAULT,
            preferred_element_type=jnp.float32,
        )
        t2s = t2s_ref[...]

        e = t2s - cross2
        ep = jnp.min(e.reshape(TN, LT, 128), axis=1)
        rowbuf_ref[...] = x2s + ep

        colpart = jnp.min(x2s - cross2, axis=0, keepdims=True)

        @pl.when(i == 0)
        def _():
            colacc_ref[...] = colpart

        @pl.when(i > 0)
        def _():
            colacc_ref[...] = jnp.minimum(colacc_ref[...], colpart)

        @pl.when(i == NI - 1)
        def _():
            colmin = t2s_ref[...] + colacc_ref[...]
            col_sqrt = jnp.sqrt(jnp.maximum(colmin, _EPS))
            out_ref[...] += jnp.sum(col_sqrt) / (M * B)

        @pl.when(last)
        def _():
            rmin = jnp.min(x2s + ep, axis=1, keepdims=True)
            out_ref[...] += jnp.sum(jnp.sqrt(jnp.maximum(rmin, _EPS))) / (N * B)

    return _chamfer_kernel, NI


def _chamfer(x, tt, interpret=False):
    B, N, D = x.shape
    M = tt.shape[2]
    TN = 512
    kern, NI = _make_kernel(B, N, M, D, TN)
    out = pl.pallas_call(
        kern,
        grid=(B, NI),
        in_specs=[
            pl.BlockSpec((1, TN, D), lambda b, i: (b, i, 0)),
            pl.BlockSpec((1, D, M), lambda b, i: (b, 0, 0)),
        ],
        out_specs=pl.BlockSpec((1, 1), lambda b, i: (0, 0)),
        out_shape=jax.ShapeDtypeStruct((1, 1), jnp.float32),
        scratch_shapes=[
            pltpu.VMEM((1, M), jnp.float32),
            pltpu.VMEM((1, M), jnp.float32),
            pltpu.VMEM((TN, 128), jnp.float32),
        ],
        interpret=interpret,
    )(x, tt)
    return out[0, 0]


@jax.jit
def _chamfer_jit(x, tt):
    return _chamfer(x, tt)


def kernel(x, target):
    tt = jnp.swapaxes(target, 1, 2)
    return _chamfer_jit(x, tt)


# speedup vs baseline: 2.1240x; 2.1240x over previous
import jax
import jax.numpy as jnp
from jax.experimental import pallas as pl
from jax.experimental.pallas import tpu as pltpu

_EPS = 1e-12


def _make_kernel(B, N, M, D, TN):
    NI = N // TN

    def _chamfer_kernel(x_ref, t_ref, out_ref, colacc_ref, t2s_ref):
        b = pl.program_id(0)
        i = pl.program_id(1)
        xb = x_ref[0]

        @pl.when(i == 0)
        def _():
            tb0 = t_ref[0]
            t2s_ref[...] = jnp.sum(tb0 * tb0, axis=0, keepdims=True)

        tb = t_ref[0]
        xb2 = xb + xb
        x2s = jnp.sum(xb * xb, axis=1, keepdims=True)

        cross2 = jax.lax.dot_general(
            xb2, tb, (((1,), (0,)), ((), ())),
            precision=jax.lax.Precision.DEFAULT,
            preferred_element_type=jnp.float32,
        )
        t2s = t2s_ref[...]

        rowpart = jnp.min(t2s - cross2, axis=1, keepdims=True)
        rowmin = x2s + rowpart
        row_contrib = jnp.sum(jnp.sqrt(jnp.maximum(rowmin, _EPS))) / (N * B)

        colpart = jnp.min(x2s - cross2, axis=0, keepdims=True)

        @pl.when(jnp.logical_and(b == 0, i == 0))
        def _():
            out_ref[...] = jnp.zeros_like(out_ref)

        @pl.when(i == 0)
        def _():
            colacc_ref[...] = colpart

        @pl.when(i > 0)
        def _():
            colacc_ref[...] = jnp.minimum(colacc_ref[...], colpart)

        out_ref[...] += row_contrib

        @pl.when(i == NI - 1)
        def _():
            colmin = t2s_ref[...] + colacc_ref[...]
            col_sqrt = jnp.sqrt(jnp.maximum(colmin, _EPS))
            out_ref[...] += jnp.sum(col_sqrt) / (M * B)

    return _chamfer_kernel, NI


def _chamfer(x, tt, interpret=False):
    B, N, D = x.shape
    M = tt.shape[2]
    TN = 512
    kern, NI = _make_kernel(B, N, M, D, TN)
    out = pl.pallas_call(
        kern,
        grid=(B, NI),
        in_specs=[
            pl.BlockSpec((1, TN, D), lambda b, i: (b, i, 0)),
            pl.BlockSpec((1, D, M), lambda b, i: (b, 0, 0)),
        ],
        out_specs=pl.BlockSpec((1, 1), lambda b, i: (0, 0)),
        out_shape=jax.ShapeDtypeStruct((1, 1), jnp.float32),
        scratch_shapes=[
            pltpu.VMEM((1, M), jnp.float32),
            pltpu.VMEM((1, M), jnp.float32),
        ],
        interpret=interpret,
    )(x, tt)
    return out[0, 0]


@jax.jit
def _chamfer_jit(x, tt):
    return _chamfer(x, tt)


def kernel(x, target):
    tt = jnp.swapaxes(target, 1, 2)
    return _chamfer_jit(x, tt)
